# baseline (speedup 1.0000x reference)
import math

import jax
import jax.numpy as jnp
from jax import lax
from jax.experimental import pallas as pl
from jax.experimental.pallas import tpu as pltpu

_B, _C, _NH, _D = 2, 768, 12, 64
_L = 1024
_BLK = 64
_NBLK = _L // _BLK
_G = 3 * _NH
_SCALE = 1.0 / math.sqrt(_D)

def _bf(t):
    return t.astype(jnp.bfloat16)


def _qkv_body(x_ref, w_ref, o_ref):
    xb = _bf(x_ref[0])
    for g in range(12):
        wb = _bf(w_ref[0, g])
        o_ref[0, g] = lax.dot_general(
            xb, wb, (((0,), (1,)), ((), ())),
            preferred_element_type=jnp.float32)


def _select_body(q_ref, k_ref, i1_ref, i2_ref):
    r = lax.broadcasted_iota(jnp.int32, (_NBLK, _L), 0)
    c = lax.broadcasted_iota(jnp.int32, (_NBLK, _L), 1) // _BLK
    pool = jnp.where(r == c, 1.0 / _BLK, 0.0).astype(jnp.float32)
    ri = lax.broadcasted_iota(jnp.int32, (_NBLK, _NBLK), 0)
    hs = list(range(4))
    qm_l = [lax.dot_general(pool, q_ref[0, hh], (((1,), (0,)), ((), ())),
                            preferred_element_type=jnp.float32,
                            precision=lax.Precision.HIGHEST) for hh in hs]
    km_l = [lax.dot_general(pool, k_ref[0, hh], (((1,), (0,)), ((), ())),
                            preferred_element_type=jnp.float32,
                            precision=lax.Precision.HIGHEST) for hh in hs]
    sc_l = [lax.dot_general(_bf(km), _bf(qm), (((1,), (1,)), ((), ())),
                            preferred_element_type=jnp.float32) * _SCALE
            for km, qm in zip(km_l, qm_l)]
    m1_l = [jnp.max(s, axis=0, keepdims=True) for s in sc_l]
    i1_l = [jnp.min(jnp.where(s == m1, ri, _NBLK), axis=0, keepdims=True)
            for s, m1 in zip(sc_l, m1_l)]
    mk_l = [jnp.where(ri == i1v, -jnp.inf, s) for s, i1v in zip(sc_l, i1_l)]
    m2_l = [jnp.max(s, axis=0, keepdims=True) for s in mk_l]
    i2_l = [jnp.min(jnp.where(s == m2, ri, _NBLK), axis=0, keepdims=True)
            for s, m2 in zip(mk_l, m2_l)]
    for hh in hs:
        i1_ref[hh] = i1_l[hh]
        i2_ref[hh] = i2_l[hh]


def _attn_body(i1_ref, i2_ref, q_ref, k_ref, v_ref, plw_ref, plb_ref, o_ref):
    p = pl.program_id(0)
    plw = plw_ref[...]
    plb = plb_ref[...]

    def _fsoftmax(t):
        e = jnp.exp(t)
        return e / jnp.sum(e, axis=-1, keepdims=True)

    for hh in range(2):
        bh = (p // 6) * _NH + (p % 6) * 2 + hh
        q = q_ref[0, hh]
        k = k_ref[0, hh]
        v = v_ref[0, hh]
        ck = _fsoftmax(k)
        cq = _fsoftmax(q)
        kv_tot = lax.dot_general(_bf(ck), _bf(v), (((0,), (0,)), ((), ())),
                                 preferred_element_type=jnp.float32)
        z_tot = jnp.sum(ck, axis=0, keepdims=True)
        qb = _bf(q)
        cqb = _bf(cq)

        _GRP = 4
        for g0 in range(0, _NBLK, _GRP):
            ids = list(range(g0, g0 + _GRP))
            js = [(i1_ref[bh, 0, i], i2_ref[bh, 0, i]) for i in ids]
            kss = [jnp.concatenate(
                [k_ref[0, hh, pl.ds(j1 * _BLK, _BLK), :],
                 k_ref[0, hh, pl.ds(j2 * _BLK, _BLK), :]], axis=0)
                for j1, j2 in js]
            vss = [jnp.concatenate(
                [v_ref[0, hh, pl.ds(j1 * _BLK, _BLK), :],
                 v_ref[0, hh, pl.ds(j2 * _BLK, _BLK), :]], axis=0)
                for j1, j2 in js]
            logit_l = [lax.dot_general(qb[i * _BLK:(i + 1) * _BLK, :], _bf(ks),
                                       (((1,), (1,)), ((), ())),
                                       preferred_element_type=jnp.float32)
                       * _SCALE for i, ks in zip(ids, kss)]
            m_l = [jnp.max(lg, axis=-1, keepdims=True) for lg in logit_l]
            pe_l = [jnp.exp(lg - m) for lg, m in zip(logit_l, m_l)]
            attn_l = [pe / jnp.sum(pe, axis=-1, keepdims=True) for pe in pe_l]
            o_s_l = [lax.dot_general(_bf(attn), _bf(vs),
                                     (((1,), (0,)), ((), ())),
                                     preferred_element_type=jnp.float32)
                     for attn, vs in zip(attn_l, vss)]
            ck_sel_l = [_fsoftmax(ks) for ks in kss]
            kv_q_l = [kv_tot - lax.dot_general(_bf(cks), _bf(vs),
                                               (((0,), (0,)), ((), ())),
                                               preferred_element_type=jnp.float32)
                      for cks, vs in zip(ck_sel_l, vss)]
            z_q_l = [z_tot - jnp.sum(cks, axis=0, keepdims=True)
                     for cks in ck_sel_l]
            num_l = [lax.dot_general(cqb[i * _BLK:(i + 1) * _BLK, :], _bf(kv_q),
                                     (((1,), (0,)), ((), ())),
                                     preferred_element_type=jnp.float32)
                     for i, kv_q in zip(ids, kv_q_l)]
            den_l = [jnp.sum(cqb[i * _BLK:(i + 1) * _BLK, :].astype(jnp.float32)
                             * _bf(z_q).astype(jnp.float32),
                             axis=-1, keepdims=True) + 1e-6
                     for i, z_q in zip(ids, z_q_l)]
            for i, o_s, num, den in zip(ids, o_s_l, num_l, den_l):
                o_blk = o_s + plb + lax.dot_general(
                    _bf(num / den), _bf(plw), (((1,), (1,)), ((), ())),
                    preferred_element_type=jnp.float32)
                o_ref[0, hh, i * _BLK:(i + 1) * _BLK, :] = o_blk


def _outproj_body(w_ref, o_ref, y_ref, ot_ref):
    for h in range(_NH):
        ot_ref[h * _D:(h + 1) * _D, :] = _bf(jnp.transpose(o_ref[0, h], (1, 0)))
    y_ref[0] = lax.dot_general(_bf(w_ref[...]), ot_ref[...],
                               (((1,), (0,)), ((), ())),
                               preferred_element_type=jnp.float32)


def kernel(x, qkv_w, out_w, proj_l_w, proj_l_b):
    b, c, h, w = x.shape
    assert (b, c, h * w) == (_B, _C, _L)
    xf = x.reshape(_B, _C, _L)

    qkvW12 = qkv_w.reshape(_G // 12, 12, _D, _C)
    qkvT = pl.pallas_call(
        _qkv_body,
        grid=(_B, _G // 12),
        in_specs=[
            pl.BlockSpec((1, _C, _L), lambda bb, g: (bb, 0, 0)),
            pl.BlockSpec((1, 12, _D, _C), lambda bb, g: (g, 0, 0, 0)),
        ],
        out_specs=pl.BlockSpec((1, 12, _L, _D), lambda bb, g: (bb, g, 0, 0)),
        out_shape=jax.ShapeDtypeStruct((_B, _G, _L, _D), jnp.float32),
    )(xf, qkvW12)

    i1, i2 = pl.pallas_call(
        _select_body,
        grid=(_B * _NH // 2,),
        in_specs=[
            pl.BlockSpec((1, 4, _L, _D), lambda s: (s // 3, s % 3, 0, 0)),
            pl.BlockSpec((1, 4, _L, _D), lambda s: (s // 3, 3 + s % 3, 0, 0)),
        ],
        out_specs=[
            pl.BlockSpec((4, 1, _NBLK), lambda s: (s, 0, 0)),
            pl.BlockSpec((4, 1, _NBLK), lambda s: (s, 0, 0)),
        ],
        out_shape=[
            jax.ShapeDtypeStruct((_B * _NH, 1, _NBLK), jnp.int32),
            jax.ShapeDtypeStruct((_B * _NH, 1, _NBLK), jnp.int32),
        ],
    )(qkvT, qkvT)

    o_heads = pl.pallas_call(
        _attn_body,
        grid=(_B * _NH // 2,),
        in_specs=[
            pl.BlockSpec(memory_space=pltpu.SMEM),
            pl.BlockSpec(memory_space=pltpu.SMEM),
            pl.BlockSpec((1, 2, _L, _D), lambda pp: (pp // 6, pp % 6, 0, 0)),
            pl.BlockSpec((1, 2, _L, _D), lambda pp: (pp // 6, 6 + pp % 6, 0, 0)),
            pl.BlockSpec((1, 2, _L, _D),
                         lambda pp: (pp // 6, 12 + pp % 6, 0, 0)),
            pl.BlockSpec((_D, _D), lambda pp: (0, 0)),
            pl.BlockSpec((1, _D), lambda pp: (0, 0)),
        ],
        out_specs=pl.BlockSpec((1, 2, _L, _D),
                               lambda pp: (pp // 6, pp % 6, 0, 0)),
        out_shape=jax.ShapeDtypeStruct((_B, _NH, _L, _D), jnp.float32),
    )(i1, i2, qkvT, qkvT, qkvT, proj_l_w, proj_l_b.reshape(1, _D))

    y = pl.pallas_call(
        _outproj_body,
        grid=(_B,),
        in_specs=[
            pl.BlockSpec((_C, _C), lambda bb: (0, 0)),
            pl.BlockSpec((1, _NH, _L, _D), lambda bb: (bb, 0, 0, 0)),
        ],
        out_specs=pl.BlockSpec((1, _C, _L), lambda bb: (bb, 0, 0)),
        out_shape=jax.ShapeDtypeStruct((_B, _C, _L), jnp.float32),
        scratch_shapes=[pltpu.VMEM((_C, _L), jnp.bfloat16)],
    )(out_w, o_heads)

    return y.reshape(_B, _C, h, w)


# speedup vs baseline: 1.1049x; 1.1049x over previous
import math

import jax
import jax.numpy as jnp
from jax import lax
from jax.experimental import pallas as pl
from jax.experimental.pallas import tpu as pltpu

_B, _C, _NH, _D = 2, 768, 12, 64
_L = 1024
_BLK = 64
_NBLK = _L // _BLK
_G = 3 * _NH
_SCALE = 1.0 / math.sqrt(_D)

def _bf(t):
    return t.astype(jnp.bfloat16)


def _qkv_body(x_ref, w_ref, o_ref):
    xb = _bf(x_ref[0])
    for g in range(12):
        wb = _bf(w_ref[0, g])
        o_ref[0, g] = lax.dot_general(
            xb, wb, (((0,), (1,)), ((), ())),
            preferred_element_type=jnp.float32)


def _select_body(q_ref, k_ref, i1_ref, i2_ref):
    r = lax.broadcasted_iota(jnp.int32, (_NBLK, _L), 0)
    c = lax.broadcasted_iota(jnp.int32, (_NBLK, _L), 1) // _BLK
    pool = jnp.where(r == c, 1.0 / _BLK, 0.0).astype(jnp.float32)
    ri = lax.broadcasted_iota(jnp.int32, (_NBLK, _NBLK), 0)
    hs = list(range(4))
    qm_l = [lax.dot_general(pool, q_ref[0, hh], (((1,), (0,)), ((), ())),
                            preferred_element_type=jnp.float32,
                            precision=lax.Precision.HIGHEST) for hh in hs]
    km_l = [lax.dot_general(pool, k_ref[0, hh], (((1,), (0,)), ((), ())),
                            preferred_element_type=jnp.float32,
                            precision=lax.Precision.HIGHEST) for hh in hs]
    sc_l = [lax.dot_general(_bf(km), _bf(qm), (((1,), (1,)), ((), ())),
                            preferred_element_type=jnp.float32) * _SCALE
            for km, qm in zip(km_l, qm_l)]
    m1_l = [jnp.max(s, axis=0, keepdims=True) for s in sc_l]
    i1_l = [jnp.min(jnp.where(s == m1, ri, _NBLK), axis=0, keepdims=True)
            for s, m1 in zip(sc_l, m1_l)]
    mk_l = [jnp.where(ri == i1v, -jnp.inf, s) for s, i1v in zip(sc_l, i1_l)]
    m2_l = [jnp.max(s, axis=0, keepdims=True) for s in mk_l]
    i2_l = [jnp.min(jnp.where(s == m2, ri, _NBLK), axis=0, keepdims=True)
            for s, m2 in zip(mk_l, m2_l)]
    for hh in hs:
        i1_ref[hh] = i1_l[hh]
        i2_ref[hh] = i2_l[hh]


def _attn_body(i1_ref, i2_ref, q_ref, k_ref, v_ref, plw_ref, plb_ref, o_ref):
    p = pl.program_id(0)
    plw = plw_ref[...]
    plb = plb_ref[...]

    def _fsoftmax(t):
        e = jnp.exp(t)
        return e / jnp.sum(e, axis=-1, keepdims=True)

    for hh in range(2):
        bh = (p // 6) * _NH + (p % 6) * 2 + hh
        q = q_ref[0, hh]
        k = k_ref[0, hh]
        v = v_ref[0, hh]
        ck = _fsoftmax(k)
        cq = _fsoftmax(q)
        kv_tot = lax.dot_general(_bf(ck), _bf(v), (((0,), (0,)), ((), ())),
                                 preferred_element_type=jnp.float32)
        z_tot = jnp.sum(ck, axis=0, keepdims=True)
        qb = _bf(q)
        cqb = _bf(cq)

        _GRP = 4
        for g0 in range(0, _NBLK, _GRP):
            ids = list(range(g0, g0 + _GRP))
            js = [(i1_ref[bh, 0, i], i2_ref[bh, 0, i]) for i in ids]
            kss = [jnp.concatenate(
                [k_ref[0, hh, pl.ds(j1 * _BLK, _BLK), :],
                 k_ref[0, hh, pl.ds(j2 * _BLK, _BLK), :]], axis=0)
                for j1, j2 in js]
            vss = [jnp.concatenate(
                [v_ref[0, hh, pl.ds(j1 * _BLK, _BLK), :],
                 v_ref[0, hh, pl.ds(j2 * _BLK, _BLK), :]], axis=0)
                for j1, j2 in js]
            logit_l = [lax.dot_general(qb[i * _BLK:(i + 1) * _BLK, :], _bf(ks),
                                       (((1,), (1,)), ((), ())),
                                       preferred_element_type=jnp.float32)
                       * _SCALE for i, ks in zip(ids, kss)]
            m_l = [jnp.max(lg, axis=-1, keepdims=True) for lg in logit_l]
            pe_l = [jnp.exp(lg - m) for lg, m in zip(logit_l, m_l)]
            attn_l = [pe / jnp.sum(pe, axis=-1, keepdims=True) for pe in pe_l]
            o_s_l = [lax.dot_general(_bf(attn), _bf(vs),
                                     (((1,), (0,)), ((), ())),
                                     preferred_element_type=jnp.float32)
                     for attn, vs in zip(attn_l, vss)]
            ck_sel_l = [_fsoftmax(ks) for ks in kss]
            kv_q_l = [kv_tot - lax.dot_general(_bf(cks), _bf(vs),
                                               (((0,), (0,)), ((), ())),
                                               preferred_element_type=jnp.float32)
                      for cks, vs in zip(ck_sel_l, vss)]
            z_q_l = [z_tot - jnp.sum(cks, axis=0, keepdims=True)
                     for cks in ck_sel_l]
            num_l = [lax.dot_general(cqb[i * _BLK:(i + 1) * _BLK, :], _bf(kv_q),
                                     (((1,), (0,)), ((), ())),
                                     preferred_element_type=jnp.float32)
                     for i, kv_q in zip(ids, kv_q_l)]
            den_l = [jnp.sum(cqb[i * _BLK:(i + 1) * _BLK, :].astype(jnp.float32)
                             * _bf(z_q).astype(jnp.float32),
                             axis=-1, keepdims=True) + 1e-6
                     for i, z_q in zip(ids, z_q_l)]
            for i, o_s, num, den in zip(ids, o_s_l, num_l, den_l):
                o_blk = o_s + plb + lax.dot_general(
                    _bf(num / den), _bf(plw), (((1,), (1,)), ((), ())),
                    preferred_element_type=jnp.float32)
                o_ref[0, hh, i * _BLK:(i + 1) * _BLK, :] = o_blk


def _outproj_body(w_ref, o_ref, y_ref, ot_ref):
    for h in range(_NH):
        ot_ref[h * _D:(h + 1) * _D, :] = _bf(jnp.transpose(o_ref[0, h], (1, 0)))
    y_ref[0] = lax.dot_general(_bf(w_ref[...]), ot_ref[...],
                               (((1,), (0,)), ((), ())),
                               preferred_element_type=jnp.float32)


def kernel(x, qkv_w, out_w, proj_l_w, proj_l_b):
    b, c, h, w = x.shape
    assert (b, c, h * w) == (_B, _C, _L)
    xf = x.reshape(_B, _C, _L)

    qkvW12 = qkv_w.reshape(_G // 12, 12, _D, _C)
    qkvT = pl.pallas_call(
        _qkv_body,
        grid=(_B, _G // 12),
        in_specs=[
            pl.BlockSpec((1, _C, _L), lambda bb, g: (bb, 0, 0)),
            pl.BlockSpec((1, 12, _D, _C), lambda bb, g: (g, 0, 0, 0)),
        ],
        out_specs=pl.BlockSpec((1, 12, _L, _D), lambda bb, g: (bb, g, 0, 0)),
        out_shape=jax.ShapeDtypeStruct((_B, _G, _L, _D), jnp.float32),
    )(xf, qkvW12)

    i1, i2 = pl.pallas_call(
        _select_body,
        grid=(_B * _NH // 4,),
        in_specs=[
            pl.BlockSpec((1, 4, _L, _D), lambda s: (s // 3, s % 3, 0, 0)),
            pl.BlockSpec((1, 4, _L, _D), lambda s: (s // 3, 3 + s % 3, 0, 0)),
        ],
        out_specs=[
            pl.BlockSpec((4, 1, _NBLK), lambda s: (s, 0, 0)),
            pl.BlockSpec((4, 1, _NBLK), lambda s: (s, 0, 0)),
        ],
        out_shape=[
            jax.ShapeDtypeStruct((_B * _NH, 1, _NBLK), jnp.int32),
            jax.ShapeDtypeStruct((_B * _NH, 1, _NBLK), jnp.int32),
        ],
    )(qkvT, qkvT)

    o_heads = pl.pallas_call(
        _attn_body,
        grid=(_B * _NH // 2,),
        in_specs=[
            pl.BlockSpec(memory_space=pltpu.SMEM),
            pl.BlockSpec(memory_space=pltpu.SMEM),
            pl.BlockSpec((1, 2, _L, _D), lambda pp: (pp // 6, pp % 6, 0, 0)),
            pl.BlockSpec((1, 2, _L, _D), lambda pp: (pp // 6, 6 + pp % 6, 0, 0)),
            pl.BlockSpec((1, 2, _L, _D),
                         lambda pp: (pp // 6, 12 + pp % 6, 0, 0)),
            pl.BlockSpec((_D, _D), lambda pp: (0, 0)),
            pl.BlockSpec((1, _D), lambda pp: (0, 0)),
        ],
        out_specs=pl.BlockSpec((1, 2, _L, _D),
                               lambda pp: (pp // 6, pp % 6, 0, 0)),
        out_shape=jax.ShapeDtypeStruct((_B, _NH, _L, _D), jnp.float32),
    )(i1, i2, qkvT, qkvT, qkvT, proj_l_w, proj_l_b.reshape(1, _D))

    y = pl.pallas_call(
        _outproj_body,
        grid=(_B,),
        in_specs=[
            pl.BlockSpec((_C, _C), lambda bb: (0, 0)),
            pl.BlockSpec((1, _NH, _L, _D), lambda bb: (bb, 0, 0, 0)),
        ],
        out_specs=pl.BlockSpec((1, _C, _L), lambda bb: (bb, 0, 0)),
        out_shape=jax.ShapeDtypeStruct((_B, _C, _L), jnp.float32),
        scratch_shapes=[pltpu.VMEM((_C, _L), jnp.bfloat16)],
    )(out_w, o_heads)

    return y.reshape(_B, _C, h, w)
